# Initial kernel scaffold; baseline (speedup 1.0000x reference)
import jax
import jax.numpy as jnp
from jax.experimental import pallas as pl


def kernel(ids, num_samples, adj_info):
    raise NotImplementedError("write your pallas kernel here")


# speedup vs baseline: 1.3167x; 1.3167x over previous
import jax
import jax.numpy as jnp
from jax import lax
from jax.experimental import pallas as pl
from jax.experimental.pallas import tpu as pltpu
from jax.experimental.pallas import tpu_sc as plsc

_NC = 2
_NS = 16
_NW = _NC * _NS
_L = 16

_B = 65536
_K = 64
_S = 32
_BPW = _B // _NW
_CHUNK = 256
_NCHUNK = _BPW // _CHUNK


def _sc_body(adj_ref, ids_ref, pos_ref, out_ref,
             ids_v, pos_v, buf_a, buf_b, out_a, out_b,
             gsem_a, gsem_b, ssem_a, ssem_b):
    wid = lax.axis_index("s") * _NC + lax.axis_index("c")
    base = wid * _BPW
    pltpu.sync_copy(ids_ref.at[pl.ds(base, _BPW)], ids_v)
    pltpu.sync_copy(pos_ref, pos_v)
    pv0 = pos_v[pl.ds(0, _L)]
    pv1 = pos_v[pl.ds(_L, _L)]

    bufs = (buf_a, buf_b)
    outs = (out_a, out_b)
    gsems = (gsem_a, gsem_b)
    ssems = (ssem_a, ssem_b)

    def start_gather(c):
        return pltpu.async_copy(
            adj_ref.at[ids_v.at[pl.ds(c * _CHUNK, _CHUNK)]],
            bufs[c % 2], gsems[c % 2])

    g = [start_gather(0), None]
    s = [None, None]
    for c in range(_NCHUNK):
        if c + 1 < _NCHUNK:
            g[(c + 1) % 2] = start_gather(c + 1)
        g[c % 2].wait()
        if c >= 2:
            s[c % 2].wait()
        buf = bufs[c % 2]
        ob = outs[c % 2]

        def row_body(r, carry, buf=buf, ob=ob):
            rvec = jnp.full((_L,), r, jnp.int32)
            ob[pl.ds(r * _S, _L)] = plsc.load_gather(buf, [rvec, pv0])
            ob[pl.ds(r * _S + _L, _L)] = plsc.load_gather(buf, [rvec, pv1])
            return carry

        lax.fori_loop(0, _CHUNK, row_body, 0)
        s[c % 2] = pltpu.async_copy(
            ob, out_ref.at[pl.ds((base + c * _CHUNK) * _S, _CHUNK * _S)],
            ssems[c % 2])
    s[0].wait()
    s[1].wait()


def _sc_gather(adj, ids, pos):
    mesh = plsc.VectorSubcoreMesh(
        core_axis_name="c", subcore_axis_name="s",
        num_cores=_NC, num_subcores=_NS)
    i32 = jnp.int32
    return pl.kernel(
        _sc_body,
        out_type=jax.ShapeDtypeStruct((_B * _S,), i32),
        mesh=mesh,
        compiler_params=pltpu.CompilerParams(
            needs_layout_passes=False, use_tc_tiling_on_sc=False),
        scratch_types=[
            pltpu.VMEM((_BPW,), i32),
            pltpu.VMEM((_S,), i32),
            pltpu.VMEM((_CHUNK, _K), i32),
            pltpu.VMEM((_CHUNK, _K), i32),
            pltpu.VMEM((_CHUNK * _S,), i32),
            pltpu.VMEM((_CHUNK * _S,), i32),
            pltpu.SemaphoreType.DMA,
            pltpu.SemaphoreType.DMA,
            pltpu.SemaphoreType.DMA,
            pltpu.SemaphoreType.DMA,
        ],
    )(adj, ids, pos)


def kernel(ids, num_samples, adj_info):
    k = adj_info.shape[1]
    perm = jax.random.permutation(jax.random.key(42), k)
    cols = perm[jnp.minimum(jnp.arange(_S), num_samples - 1)].astype(jnp.int32)
    out = _sc_gather(adj_info.astype(jnp.int32), ids.astype(jnp.int32), cols)
    return out.reshape(_B, _S).astype(adj_info.dtype)


# speedup vs baseline: 3.0249x; 2.2974x over previous
import jax
import jax.numpy as jnp
from jax import lax
from jax.experimental import pallas as pl
from jax.experimental.pallas import tpu as pltpu
from jax.experimental.pallas import tpu_sc as plsc

_NC = 2
_NS = 16
_NW = _NC * _NS
_L = 16

_N = 100000
_B = 65536
_K = 64
_S = 32
_CHUNK = 4096
_NCHUNK = _B // _CHUNK


def _sc_body(adjt_ref, ids_ref, pos_ref, out_ref,
             col_v, pos_v, ids_a, ids_b, out_a, out_b,
             csem, gsem_a, gsem_b, ssem_a, ssem_b):
    wid = lax.axis_index("s") * _NC + lax.axis_index("c")
    pltpu.sync_copy(pos_ref, pos_v)
    grp = (wid // _L) * _L
    pv = pos_v[pl.ds(grp, _L)]
    lane = lax.iota(jnp.int32, _L)
    c = jnp.sum(jnp.where(lane == wid - grp, pv, 0))

    col_dma = pltpu.async_copy(adjt_ref.at[c], col_v, csem)

    idbufs = (ids_a, ids_b)
    obufs = (out_a, out_b)
    gsems = (gsem_a, gsem_b)
    ssems = (ssem_a, ssem_b)

    def start_ids(k):
        return pltpu.async_copy(
            ids_ref.at[pl.ds(k * _CHUNK, _CHUNK)], idbufs[k % 2], gsems[k % 2])

    g = [start_ids(0), None]
    s = [None, None]
    col_dma.wait()
    for k in range(_NCHUNK):
        if k + 1 < _NCHUNK:
            g[(k + 1) % 2] = start_ids(k + 1)
        g[k % 2].wait()
        if k >= 2:
            s[k % 2].wait()
        idv = idbufs[k % 2]
        ob = obufs[k % 2]

        def grp_body(i, carry, idv=idv, ob=ob):
            idx = idv[pl.ds(i * _L, _L)]
            ob[pl.ds(i * _L, _L)] = plsc.load_gather(col_v, [idx])
            return carry

        lax.fori_loop(0, _CHUNK // _L, grp_body, 0)
        s[k % 2] = pltpu.async_copy(
            ob, out_ref.at[wid, pl.ds(k * _CHUNK, _CHUNK)], ssems[k % 2])
    s[0].wait()
    s[1].wait()


def _sc_gather(adjt, ids, pos):
    mesh = plsc.VectorSubcoreMesh(
        core_axis_name="c", subcore_axis_name="s",
        num_cores=_NC, num_subcores=_NS)
    i32 = jnp.int32
    return pl.kernel(
        _sc_body,
        out_type=jax.ShapeDtypeStruct((_S, _B), i32),
        mesh=mesh,
        compiler_params=pltpu.CompilerParams(
            needs_layout_passes=False, use_tc_tiling_on_sc=True),
        scratch_types=[
            pltpu.VMEM((_N,), i32),
            pltpu.VMEM((_S,), i32),
            pltpu.VMEM((_CHUNK,), i32),
            pltpu.VMEM((_CHUNK,), i32),
            pltpu.VMEM((_CHUNK,), i32),
            pltpu.VMEM((_CHUNK,), i32),
            pltpu.SemaphoreType.DMA,
            pltpu.SemaphoreType.DMA,
            pltpu.SemaphoreType.DMA,
            pltpu.SemaphoreType.DMA,
            pltpu.SemaphoreType.DMA,
        ],
    )(adjt, ids, pos)


def kernel(ids, num_samples, adj_info):
    k = adj_info.shape[1]
    perm = jax.random.permutation(jax.random.key(42), k)
    cols = perm[jnp.minimum(jnp.arange(_S), num_samples - 1)].astype(jnp.int32)
    out_t = _sc_gather(adj_info.T.astype(jnp.int32), ids.astype(jnp.int32),
                       cols)
    return out_t.T.astype(adj_info.dtype)
